# baseline (speedup 1.0000x reference)
import jax
import jax.numpy as jnp
from jax.experimental import pallas as pl
from jax.experimental.pallas import tpu as pltpu

N_CELLS = 360
FEAT = 16
LAT = 64
BAND = 8
BATCH = 32
CIOS = 2844

NP = 384
SEG = 368
NSEG = 18
EH = NSEG * SEG
CB = 4


def _mm(x, w):
    return jax.lax.dot_general(x, w,
                               (((1,), (0,)), ((), ())),
                               preferred_element_type=jnp.float32)


def _ln_rows(x, g, b):
    m = jnp.mean(x, axis=-1, keepdims=True)
    v = jnp.mean((x - m) ** 2, axis=-1, keepdims=True)
    return (x - m) * jax.lax.rsqrt(v + 1e-5) * g + b


def _gnn_kernel(*refs):
    it = iter(refs[:-1])
    out_ref = refs[-1]

    obs_ref = next(it)
    few = next(it)[...]; neg = next(it)[...]; neb = next(it)[...]
    cls = next(it)[...]; feb = next(it)[...]
    stk = []
    for i in range(3):
        s = {k: next(it) for k in
             ('ew1', 'ew2', 'ew3', 'eb1', 'eb2', 'eb3',
              'nw1', 'nw2', 'nw3', 'nb1', 'nb2', 'nb3', 'nng', 'nnb')}
        if i < 2:
            s['eng'] = next(it); s['enb'] = next(it)
        stk.append(s)

    x = jnp.maximum(_mm(obs_ref[...].reshape(CB * N_CELLS, FEAT), few)
                    .reshape(CB, N_CELLS, LAT) + feb, 0.0)
    m = jnp.mean(x, axis=(1, 2), keepdims=True)
    v = jnp.mean((x - m) ** 2, axis=(1, 2), keepdims=True)
    x = (x - m) * jax.lax.rsqrt(v + 1e-5) * neg + neb

    node = jnp.concatenate(
        [jnp.broadcast_to(cls, (CB, 1, LAT)), x,
         jnp.zeros((CB, NP - N_CELLS - 1, LAT), jnp.float32)],
        axis=1)

    vi = jax.lax.broadcasted_iota(jnp.int32, (NP, 1), 0).astype(jnp.float32)
    deg = jnp.where(
        vi == 0.0, float(N_CELLS),
        jnp.where(vi <= float(N_CELLS),
                  1.0 + jnp.clip(float(N_CELLS) - vi, 0.0, float(BAND))
                  + jnp.clip(vi - 1.0, 0.0, float(BAND)),
                  0.0))

    def build_h(node):
        n2d = node.reshape(CB * NP, LAT)
        pp = _mm(n2d, w1s).reshape(CB, NP, LAT)
        qp = (_mm(n2d, w1d) + s['eb1'][...]).reshape(CB, NP, LAT)
        pieces = [pp[:, 0:1, :] + qp[:, 1:1 + SEG, :],
                  pp[:, 1:1 + SEG, :] + qp[:, 0:1, :]]
        for d in range(1, BAND + 1):
            pieces.append(pp[:, d + 1:d + 1 + SEG, :] + qp[:, 1:1 + SEG, :])
        for d in range(1, BAND + 1):
            pieces.append(pp[:, 1:1 + SEG, :] + qp[:, d + 1:d + 1 + SEG, :])
        return jnp.concatenate(pieces, axis=1).reshape(CB * EH, LAT)

    def scatter(ef):
        s_band = ef[:, 0:N_CELLS, :]
        for d in range(1, BAND + 1):
            o = (1 + d) * SEG
            s_band = s_band + ef[:, o:o + N_CELLS, :]
        corr = ef[:, 9 * SEG + N_CELLS - BAND:9 * SEG + N_CELLS, :]
        for d in range(1, BAND):
            o = (1 + d) * SEG + N_CELLS - d
            corr = corr + jnp.concatenate(
                [jnp.zeros((CB, BAND - d, LAT), jnp.float32),
                 ef[:, o:o + d, :]], axis=1)
        s_band = jnp.concatenate(
            [s_band[:, 0:N_CELLS - BAND, :],
             s_band[:, N_CELLS - BAND:N_CELLS, :] - corr], axis=1)
        for d in range(1, BAND + 1):
            o = (9 + d) * SEG
            s_band = s_band + jnp.concatenate(
                [jnp.zeros((CB, d, LAT), jnp.float32),
                 ef[:, o:o + N_CELLS - d, :]], axis=1)
        hub0 = jnp.sum(ef[:, SEG:SEG + N_CELLS, :], axis=1, keepdims=True)
        return jnp.concatenate(
            [hub0, s_band,
             jnp.zeros((CB, NP - N_CELLS - 1, LAT), jnp.float32)], axis=1)

    efeat = None
    for i in range(2):
        s = stk[i]
        ew1 = s['ew1']
        w1s = ew1[0:LAT, :]; w1d = ew1[LAT:2 * LAT, :]
        h1 = build_h(node)
        if i > 0:
            h1 = h1 + _mm(efeat, ew1[2 * LAT:3 * LAT, :])
        h1 = jnp.maximum(h1, 0.0)
        h2 = jnp.maximum(_mm(h1, s['ew2'][...]) + s['eb2'][...], 0.0)
        if i == 0:
            ef2d = _mm(h2, s['ew3'][...]) + s['eb3'][...]
            agg = scatter(ef2d.reshape(CB, EH, LAT))
            efeat = _ln_rows(ef2d, s['eng'][...], s['enb'][...])
        else:
            aggh = scatter(h2.reshape(CB, EH, LAT))
            agg = (_mm(aggh.reshape(CB * NP, LAT), s['ew3'][...])
                   .reshape(CB, NP, LAT) + (deg * s['eb3'][...])[None])
            h2hub = h2.reshape(CB, EH, LAT)[:, SEG:2 * SEG, :]
            efhub = (_mm(h2hub.reshape(CB * SEG, LAT), s['ew3'][...])
                     + s['eb3'][...])
            efeat = _ln_rows(efhub, s['eng'][...], s['enb'][...])

        n2d = node.reshape(CB * NP, LAT)
        g1 = jnp.maximum(_mm(n2d, s['nw1'][0:LAT, :])
                         + _mm(agg.reshape(CB * NP, LAT),
                               s['nw1'][LAT:2 * LAT, :])
                         + s['nb1'][...], 0.0)
        g2 = jnp.maximum(_mm(g1, s['nw2'][...]) + s['nb2'][...], 0.0)
        g3 = _mm(g2, s['nw3'][...]) + s['nb3'][...]
        node = _ln_rows(g3, s['nng'][...], s['nnb'][...]).reshape(CB, NP, LAT)

    s = stk[2]
    ew1 = s['ew1']
    node0 = node[:, 0, :]
    pp = _mm(node.reshape(CB * NP, LAT), ew1[0:LAT, :]).reshape(CB, NP, LAT)
    q0 = (_mm(node0, ew1[LAT:2 * LAT, :]) + s['eb1'][...]).reshape(CB, 1, LAT)
    h1 = ((pp[:, 1:1 + SEG, :] + q0).reshape(CB * SEG, LAT)
          + _mm(efeat, ew1[2 * LAT:3 * LAT, :]))
    h1 = jnp.maximum(h1, 0.0)
    h2 = jnp.maximum(_mm(h1, s['ew2'][...]) + s['eb2'][...], 0.0)
    h2sum = jnp.sum(h2.reshape(CB, SEG, LAT)[:, 0:N_CELLS, :], axis=1)
    agg0 = _mm(h2sum, s['ew3'][...]) + float(N_CELLS) * s['eb3'][...]

    g1 = jnp.maximum(_mm(node0, s['nw1'][0:LAT, :])
                     + _mm(agg0, s['nw1'][LAT:2 * LAT, :]) + s['nb1'][...], 0.0)
    g2 = jnp.maximum(_mm(g1, s['nw2'][...]) + s['nb2'][...], 0.0)
    g3 = _mm(g2, s['nw3'][...]) + s['nb3'][...]
    node0 = _ln_rows(g3, s['nng'][...], s['nnb'][...])

    out_ref[0] = node0


def _head_kernel(nf_ref, act_ref, aew_ref, aeb_ref, nag_ref, nab_ref,
                 outw_ref, outb_ref, o_ref):
    a = _ln_rows(_mm(act_ref[...], aew_ref[...]) + aeb_ref[...],
                 nag_ref[...], nab_ref[...])
    o_ref[...] = (_mm(nf_ref[...], outw_ref[0:LAT, :])
                  + _mm(a, outw_ref[LAT:2 * LAT, :]) + outb_ref[...])


@jax.jit
def _run(obs3, actions, aew, aeb, nag, nab, outw, outb, *weights):
    grid = (BATCH // CB,)

    def _const_spec(arr):
        nd = arr.ndim
        return pl.BlockSpec(arr.shape, lambda i, _n=nd: (0,) * _n)

    specs = ([pl.BlockSpec((CB, N_CELLS, FEAT), lambda i: (i, 0, 0))]
             + [_const_spec(w) for w in weights])
    nf = pl.pallas_call(
        _gnn_kernel,
        grid=grid,
        in_specs=specs,
        out_specs=pl.BlockSpec((1, CB, LAT), lambda i: (i, 0, 0)),
        out_shape=jax.ShapeDtypeStruct((BATCH // CB, CB, LAT), jnp.float32),
        compiler_params=pltpu.CompilerParams(
            dimension_semantics=("arbitrary",)),
    )(obs3, *weights).reshape(BATCH, LAT)
    return pl.pallas_call(
        _head_kernel,
        out_shape=jax.ShapeDtypeStruct((BATCH, CIOS), jnp.float32),
    )(nf, actions, aew, aeb, nag, nab, outw, outb)


def kernel(observations, actions, params, edge_index):
    p = params
    r = lambda v: v.reshape(1, LAT)
    weights = [p['fe_w'], p['ne_g'], p['ne_b'], p['cls'], r(p['fe_b'])]
    for i in range(3):
        ew = p['emlp%d_w' % i]; eb = p['emlp%d_b' % i]
        nw = p['nmlp%d_w' % i]; nb = p['nmlp%d_b' % i]
        weights += [ew[0], ew[1], ew[2], r(eb[0]), r(eb[1]), r(eb[2]),
                    nw[0], nw[1], nw[2], r(nb[0]), r(nb[1]), r(nb[2]),
                    r(p['nn%d_g' % i]), r(p['nn%d_b' % i])]
        if i < 2:
            weights += [r(p['en%d_g' % i]), r(p['en%d_b' % i])]
    out = _run(observations.reshape(BATCH, N_CELLS, FEAT), actions,
               p['ae_w'], r(p['ae_b']), r(p['na_g']), r(p['na_b']),
               p['out_w'], p['out_b'].reshape(1, CIOS), *weights)
    return out[:, :, None]


# speedup vs baseline: 1.0272x; 1.0272x over previous
import jax
import jax.numpy as jnp
from jax.experimental import pallas as pl
from jax.experimental.pallas import tpu as pltpu

N_CELLS = 360
FEAT = 16
LAT = 64
BAND = 8
BATCH = 32
CIOS = 2844

NP = 384
SEG = 368
NSEG = 18
EH = NSEG * SEG
CB = 4


def _mm(x, w):
    return jax.lax.dot_general(x, w,
                               (((1,), (0,)), ((), ())),
                               preferred_element_type=jnp.float32)


def _ln_rows(x, g, b):
    m = jnp.mean(x, axis=-1, keepdims=True)
    v = jnp.mean((x - m) ** 2, axis=-1, keepdims=True)
    return (x - m) * jax.lax.rsqrt(v + 1e-5) * g + b


def _gnn_kernel(*refs):
    it = iter(refs[:-3])
    out_ref, h_ref, agg_ref = refs[-3:]

    obs_ref = next(it)
    few = next(it)[...]; neg = next(it)[...]; neb = next(it)[...]
    cls = next(it)[...]; feb = next(it)[...]
    stk = []
    for i in range(3):
        s = {k: next(it) for k in
             ('ew1', 'ew2', 'ew3', 'eb1', 'eb2', 'eb3',
              'nw1', 'nw2', 'nw3', 'nb1', 'nb2', 'nb3', 'nng', 'nnb')}
        if i < 2:
            s['eng'] = next(it); s['enb'] = next(it)
        stk.append(s)

    x = jnp.maximum(_mm(obs_ref[...].reshape(CB * N_CELLS, FEAT), few)
                    .reshape(CB, N_CELLS, LAT) + feb, 0.0)
    m = jnp.mean(x, axis=(1, 2), keepdims=True)
    v = jnp.mean((x - m) ** 2, axis=(1, 2), keepdims=True)
    x = (x - m) * jax.lax.rsqrt(v + 1e-5) * neg + neb

    agg_ref[:, 0:1, :] = jnp.broadcast_to(cls, (CB, 1, LAT))
    agg_ref[:, 1:N_CELLS + 1, :] = x
    agg_ref[:, N_CELLS + 1:, :] = jnp.zeros((CB, NP - N_CELLS - 1, LAT),
                                            jnp.float32)
    node = agg_ref[...]

    vi = jax.lax.broadcasted_iota(jnp.int32, (NP, 1), 0).astype(jnp.float32)
    deg = jnp.where(
        vi == 0.0, float(N_CELLS),
        jnp.where(vi <= float(N_CELLS),
                  1.0 + jnp.clip(float(N_CELLS) - vi, 0.0, float(BAND))
                  + jnp.clip(vi - 1.0, 0.0, float(BAND)),
                  0.0))

    def build_h(node):
        n2d = node.reshape(CB * NP, LAT)
        pp = _mm(n2d, w1s).reshape(CB, NP, LAT)
        qp = (_mm(n2d, w1d) + s['eb1'][...]).reshape(CB, NP, LAT)
        pieces = [pp[:, 0:1, :] + qp[:, 1:1 + SEG, :],
                  pp[:, 1:1 + SEG, :] + qp[:, 0:1, :]]
        for d in range(1, BAND + 1):
            pieces.append(pp[:, d + 1:d + 1 + SEG, :] + qp[:, 1:1 + SEG, :])
        for d in range(1, BAND + 1):
            pieces.append(pp[:, 1:1 + SEG, :] + qp[:, d + 1:d + 1 + SEG, :])
        return jnp.concatenate(pieces, axis=1).reshape(CB * EH, LAT)

    def scatter(ef):
        s_band = ef[:, 0:N_CELLS, :]
        for d in range(1, BAND + 1):
            o = (1 + d) * SEG
            s_band = s_band + ef[:, o:o + N_CELLS, :]
        agg_ref[:, 0:1, :] = jnp.sum(ef[:, SEG:SEG + N_CELLS, :], axis=1,
                                     keepdims=True)
        agg_ref[:, 1:N_CELLS + 1, :] = s_band
        agg_ref[:, N_CELLS + 1:, :] = jnp.zeros((CB, NP - N_CELLS - 1, LAT),
                                                jnp.float32)
        for d in range(1, BAND + 1):
            o = (1 + d) * SEG + N_CELLS - d
            agg_ref[:, N_CELLS + 1 - d:N_CELLS + 1, :] += -ef[:, o:o + d, :]
        for d in range(1, BAND + 1):
            o = (9 + d) * SEG
            agg_ref[:, d + 1:d + 1 + N_CELLS, :] += ef[:, o:o + N_CELLS, :]
        return agg_ref[...]

    efeat = None
    for i in range(2):
        s = stk[i]
        ew1 = s['ew1']
        w1s = ew1[0:LAT, :]; w1d = ew1[LAT:2 * LAT, :]
        h1 = build_h(node)
        if i > 0:
            h1 = h1 + _mm(efeat, ew1[2 * LAT:3 * LAT, :])
        h1 = jnp.maximum(h1, 0.0)
        h2 = jnp.maximum(_mm(h1, s['ew2'][...]) + s['eb2'][...], 0.0)
        if i == 0:
            ef2d = _mm(h2, s['ew3'][...]) + s['eb3'][...]
            agg = scatter(ef2d.reshape(CB, EH, LAT))
            efeat = _ln_rows(ef2d, s['eng'][...], s['enb'][...])
        else:
            aggh = scatter(h2.reshape(CB, EH, LAT))
            agg = (_mm(aggh.reshape(CB * NP, LAT), s['ew3'][...])
                   .reshape(CB, NP, LAT) + (deg * s['eb3'][...])[None])
            h2hub = h2.reshape(CB, EH, LAT)[:, SEG:2 * SEG, :]
            efhub = (_mm(h2hub.reshape(CB * SEG, LAT), s['ew3'][...])
                     + s['eb3'][...])
            efeat = _ln_rows(efhub, s['eng'][...], s['enb'][...])

        n2d = node.reshape(CB * NP, LAT)
        g1 = jnp.maximum(_mm(n2d, s['nw1'][0:LAT, :])
                         + _mm(agg.reshape(CB * NP, LAT),
                               s['nw1'][LAT:2 * LAT, :])
                         + s['nb1'][...], 0.0)
        g2 = jnp.maximum(_mm(g1, s['nw2'][...]) + s['nb2'][...], 0.0)
        g3 = _mm(g2, s['nw3'][...]) + s['nb3'][...]
        node = _ln_rows(g3, s['nng'][...], s['nnb'][...]).reshape(CB, NP, LAT)

    s = stk[2]
    ew1 = s['ew1']
    node0 = node[:, 0, :]
    pp = _mm(node.reshape(CB * NP, LAT), ew1[0:LAT, :]).reshape(CB, NP, LAT)
    q0 = (_mm(node0, ew1[LAT:2 * LAT, :]) + s['eb1'][...]).reshape(CB, 1, LAT)
    h1 = ((pp[:, 1:1 + SEG, :] + q0).reshape(CB * SEG, LAT)
          + _mm(efeat, ew1[2 * LAT:3 * LAT, :]))
    h1 = jnp.maximum(h1, 0.0)
    h2 = jnp.maximum(_mm(h1, s['ew2'][...]) + s['eb2'][...], 0.0)
    h2sum = jnp.sum(h2.reshape(CB, SEG, LAT)[:, 0:N_CELLS, :], axis=1)
    agg0 = _mm(h2sum, s['ew3'][...]) + float(N_CELLS) * s['eb3'][...]

    g1 = jnp.maximum(_mm(node0, s['nw1'][0:LAT, :])
                     + _mm(agg0, s['nw1'][LAT:2 * LAT, :]) + s['nb1'][...], 0.0)
    g2 = jnp.maximum(_mm(g1, s['nw2'][...]) + s['nb2'][...], 0.0)
    g3 = _mm(g2, s['nw3'][...]) + s['nb3'][...]
    node0 = _ln_rows(g3, s['nng'][...], s['nnb'][...])

    out_ref[0] = node0


def _head_kernel(nf_ref, act_ref, aew_ref, aeb_ref, nag_ref, nab_ref,
                 outw_ref, outb_ref, o_ref):
    a = _ln_rows(_mm(act_ref[...], aew_ref[...]) + aeb_ref[...],
                 nag_ref[...], nab_ref[...])
    o_ref[...] = (_mm(nf_ref[...], outw_ref[0:LAT, :])
                  + _mm(a, outw_ref[LAT:2 * LAT, :]) + outb_ref[...])


@jax.jit
def _run(obs3, actions, aew, aeb, nag, nab, outw, outb, *weights):
    grid = (BATCH // CB,)

    def _const_spec(arr):
        nd = arr.ndim
        return pl.BlockSpec(arr.shape, lambda i, _n=nd: (0,) * _n)

    specs = ([pl.BlockSpec((CB, N_CELLS, FEAT), lambda i: (i, 0, 0))]
             + [_const_spec(w) for w in weights])
    nf = pl.pallas_call(
        _gnn_kernel,
        grid=grid,
        in_specs=specs,
        out_specs=pl.BlockSpec((1, CB, LAT), lambda i: (i, 0, 0)),
        out_shape=jax.ShapeDtypeStruct((BATCH // CB, CB, LAT), jnp.float32),
        scratch_shapes=[pltpu.VMEM((CB, EH, LAT), jnp.float32),
                        pltpu.VMEM((CB, NP, LAT), jnp.float32)],
        compiler_params=pltpu.CompilerParams(
            dimension_semantics=("arbitrary",)),
    )(obs3, *weights).reshape(BATCH, LAT)
    return pl.pallas_call(
        _head_kernel,
        out_shape=jax.ShapeDtypeStruct((BATCH, CIOS), jnp.float32),
    )(nf, actions, aew, aeb, nag, nab, outw, outb)


def kernel(observations, actions, params, edge_index):
    p = params
    r = lambda v: v.reshape(1, LAT)
    weights = [p['fe_w'], p['ne_g'], p['ne_b'], p['cls'], r(p['fe_b'])]
    for i in range(3):
        ew = p['emlp%d_w' % i]; eb = p['emlp%d_b' % i]
        nw = p['nmlp%d_w' % i]; nb = p['nmlp%d_b' % i]
        weights += [ew[0], ew[1], ew[2], r(eb[0]), r(eb[1]), r(eb[2]),
                    nw[0], nw[1], nw[2], r(nb[0]), r(nb[1]), r(nb[2]),
                    r(p['nn%d_g' % i]), r(p['nn%d_b' % i])]
        if i < 2:
            weights += [r(p['en%d_g' % i]), r(p['en%d_b' % i])]
    out = _run(observations.reshape(BATCH, N_CELLS, FEAT), actions,
               p['ae_w'], r(p['ae_b']), r(p['na_g']), r(p['na_b']),
               p['out_w'], p['out_b'].reshape(1, CIOS), *weights)
    return out[:, :, None]
